# Initial kernel scaffold; baseline (speedup 1.0000x reference)
import jax
import jax.numpy as jnp
from jax.experimental import pallas as pl


def kernel(x, edge_index, W1, b1, gamma, beta, W2, b2):
    raise NotImplementedError("write your pallas kernel here")


# speedup vs baseline: 17.9901x; 17.9901x over previous
import functools

import jax
import jax.numpy as jnp
from jax import lax
from jax.experimental import pallas as pl
from jax.experimental.pallas import tpu as pltpu
from jax.experimental.pallas import tpu_sc as plsc

N = 10000
E = 320000
D = 128
DO = 16
NC = 2
NS = 16
NW = NC * NS
EB = 128
NBLK = E // EB
ZR = 80
NZ = N // ZR
L = 16
BN_EPS = 1e-5
RB = 1000

_mesh = plsc.VectorSubcoreMesh(core_axis_name="c", subcore_axis_name="s")
_sc_params = pltpu.CompilerParams(use_tc_tiling_on_sc=False)


@functools.partial(
    pl.kernel,
    out_type=jax.ShapeDtypeStruct((NC, N, L), jnp.float32),
    mesh=_mesh,
    compiler_params=_sc_params,
    scratch_types=[
        pltpu.VMEM((EB, L), jnp.float32),
        pltpu.VMEM((1, EB), jnp.int32),
        pltpu.VMEM((ZR, L), jnp.float32),
        pltpu.VMEM_SHARED((N, L), jnp.float32),
    ],
)
def _sc_count(dst_hbm, acc_hbm, ones_v, didx_v, zbuf_v, shared):
    c = lax.axis_index("c")
    s = lax.axis_index("s")
    w = s * NC + c

    @pl.loop(0, EB)
    def _(i):
        ones_v[i, :] = jnp.full((L,), 1.0, jnp.float32)

    @pl.loop(0, ZR)
    def _(i):
        zbuf_v[i, :] = jnp.zeros((L,), jnp.float32)

    @pl.loop(s, NZ, step=NS)
    def _(i):
        pltpu.sync_copy(zbuf_v, shared.at[pl.ds(i * ZR, ZR)])

    plsc.subcore_barrier()

    @pl.loop(w, NBLK, step=NW)
    def _(b):
        pltpu.sync_copy(dst_hbm.at[pl.ds(b, 1)], didx_v)
        pltpu.sync_copy(ones_v, shared.at[didx_v.at[0]], add=True)

    plsc.subcore_barrier()

    @pl.loop(s, NZ, step=NS)
    def _(i):
        pltpu.sync_copy(shared.at[pl.ds(i * ZR, ZR)],
                        acc_hbm.at[c].at[pl.ds(i * ZR, ZR)])


def _make_sc_propagate(width):
    @functools.partial(
        pl.kernel,
        out_type=jax.ShapeDtypeStruct((NC, N, width), jnp.float32),
        mesh=_mesh,
        compiler_params=_sc_params,
        scratch_types=[
            pltpu.VMEM((1, EB), jnp.int32),
            pltpu.VMEM((1, EB), jnp.int32),
            pltpu.VMEM((EB, width), jnp.float32),
            pltpu.VMEM((ZR, width), jnp.float32),
            pltpu.VMEM_SHARED((N, width), jnp.float32),
        ],
    )
    def _sc_prop(vals_hbm, src_hbm, dst_hbm, acc_hbm,
                 sidx_v, didx_v, rows_v, zbuf_v, shared):
        c = lax.axis_index("c")
        s = lax.axis_index("s")
        w = s * NC + c

        @pl.loop(0, ZR)
        def _(i):
            @pl.loop(0, width, step=L)
            def _(j):
                zbuf_v[i, pl.ds(j, L)] = jnp.zeros((L,), jnp.float32)

        @pl.loop(s, NZ, step=NS)
        def _(i):
            pltpu.sync_copy(zbuf_v, shared.at[pl.ds(i * ZR, ZR)])

        plsc.subcore_barrier()

        @pl.loop(w, NBLK, step=NW)
        def _(b):
            pltpu.sync_copy(src_hbm.at[pl.ds(b, 1)], sidx_v)
            pltpu.sync_copy(dst_hbm.at[pl.ds(b, 1)], didx_v)
            pltpu.sync_copy(vals_hbm.at[sidx_v.at[0]], rows_v)
            pltpu.sync_copy(rows_v, shared.at[didx_v.at[0]], add=True)

        plsc.subcore_barrier()

        @pl.loop(s, NZ, step=NS)
        def _(i):
            pltpu.sync_copy(shared.at[pl.ds(i * ZR, ZR)],
                            acc_hbm.at[c].at[pl.ds(i * ZR, ZR)])

    return _sc_prop


_sc_prop_d = _make_sc_propagate(D)
_sc_prop_do = _make_sc_propagate(DO)


def _tc_mm1_body(x_ref, w_ref, h_ref):
    h_ref[...] = jnp.dot(x_ref[...], w_ref[...],
                         preferred_element_type=jnp.float32)


def _tc_mm1(x, w1):
    return pl.pallas_call(
        _tc_mm1_body,
        grid=(N // RB,),
        in_specs=[
            pl.BlockSpec((RB, D), lambda i: (i, 0)),
            pl.BlockSpec((D, D), lambda i: (0, 0)),
        ],
        out_specs=pl.BlockSpec((RB, D), lambda i: (i, 0)),
        out_shape=jax.ShapeDtypeStruct((N, D), jnp.float32),
    )(x, w1)


def _tc_scale_body(degacc_ref, h1_ref, dinv_ref, h1s_ref):
    deg = degacc_ref[0] + degacc_ref[1] + 1.0
    dinv = lax.rsqrt(deg)
    dinv_ref[...] = dinv
    h1s_ref[...] = h1_ref[...] * dinv[:, :1]


def _tc_scale(degacc, h1):
    return pl.pallas_call(
        _tc_scale_body,
        grid=(N // RB,),
        in_specs=[
            pl.BlockSpec((NC, RB, L), lambda i: (0, i, 0)),
            pl.BlockSpec((RB, D), lambda i: (i, 0)),
        ],
        out_specs=[
            pl.BlockSpec((RB, L), lambda i: (i, 0)),
            pl.BlockSpec((RB, D), lambda i: (i, 0)),
        ],
        out_shape=[
            jax.ShapeDtypeStruct((N, L), jnp.float32),
            jax.ShapeDtypeStruct((N, D), jnp.float32),
        ],
    )(degacc, h1)


def _tc_mid_body(acc_ref, h1_ref, dinv_ref, b1_ref, g_ref, be_ref, w2_ref,
                 h2_ref, h2s_ref):
    dinv = dinv_ref[:, :1]
    out1 = (acc_ref[0] + acc_ref[1]) * dinv + h1_ref[...] * (dinv * dinv)
    out1 = out1 + b1_ref[...]
    act = jnp.maximum(out1, 0.0)
    scale = g_ref[...] * lax.rsqrt(jnp.float32(1.0 + BN_EPS))
    act = act * scale + be_ref[...]
    h2 = jnp.dot(act, w2_ref[...], preferred_element_type=jnp.float32)
    h2_ref[...] = h2
    h2s_ref[...] = h2 * dinv


def _tc_mid(accv, h1, dinv16, b1, gamma, beta, w2p):
    return pl.pallas_call(
        _tc_mid_body,
        grid=(N // RB,),
        in_specs=[
            pl.BlockSpec((NC, RB, D), lambda i: (0, i, 0)),
            pl.BlockSpec((RB, D), lambda i: (i, 0)),
            pl.BlockSpec((RB, L), lambda i: (i, 0)),
            pl.BlockSpec((1, D), lambda i: (0, 0)),
            pl.BlockSpec((1, D), lambda i: (0, 0)),
            pl.BlockSpec((1, D), lambda i: (0, 0)),
            pl.BlockSpec((D, DO), lambda i: (0, 0)),
        ],
        out_specs=[
            pl.BlockSpec((RB, DO), lambda i: (i, 0)),
            pl.BlockSpec((RB, DO), lambda i: (i, 0)),
        ],
        out_shape=[
            jax.ShapeDtypeStruct((N, DO), jnp.float32),
            jax.ShapeDtypeStruct((N, DO), jnp.float32),
        ],
    )(accv, h1, dinv16, b1, gamma, beta, w2p)


def _tc_fin_body(acc_ref, h2_ref, dinv_ref, b2_ref, out_ref):
    dinv = dinv_ref[:, :1]
    out = (acc_ref[0] + acc_ref[1]) * dinv + h2_ref[...] * (dinv * dinv)
    out_ref[...] = out + b2_ref[...]


def _tc_fin(acc2, h2, dinv16, b2p):
    return pl.pallas_call(
        _tc_fin_body,
        grid=(N // RB,),
        in_specs=[
            pl.BlockSpec((NC, RB, DO), lambda i: (0, i, 0)),
            pl.BlockSpec((RB, DO), lambda i: (i, 0)),
            pl.BlockSpec((RB, L), lambda i: (i, 0)),
            pl.BlockSpec((1, DO), lambda i: (0, 0)),
        ],
        out_specs=pl.BlockSpec((RB, DO), lambda i: (i, 0)),
        out_shape=jax.ShapeDtypeStruct((N, DO), jnp.float32),
    )(acc2, h2, dinv16, b2p)


def kernel(x, edge_index, W1, b1, gamma, beta, W2, b2):
    src = edge_index[0].reshape(NBLK, EB)
    dst = edge_index[1].reshape(NBLK, EB)

    w2p = jnp.zeros((D, DO), jnp.float32).at[:, :3].set(W2)
    b2p = jnp.zeros((1, DO), jnp.float32).at[0, :3].set(b2)

    degacc = _sc_count(dst)
    h1 = _tc_mm1(x, W1)
    dinv16, h1s = _tc_scale(degacc, h1)
    accv = _sc_prop_d(h1s, src, dst)
    h2, h2s = _tc_mid(accv, h1, dinv16,
                      b1.reshape(1, D), gamma.reshape(1, D),
                      beta.reshape(1, D), w2p)
    acc2 = _sc_prop_do(h2s, src, dst)
    out16 = _tc_fin(acc2, h2, dinv16, b2p)
    return out16[:, :3]
